# baseline (device time: 846857 ns/iter reference)
import jax
import jax.numpy as jnp
from jax import lax
from jax.experimental import pallas as pl
from jax.experimental.pallas import tpu as pltpu

jax.config.update("jax_compilation_cache_dir", "/tmp/jax_cache")
jax.config.update("jax_persistent_cache_min_compile_time_secs", 0.0)

N_DEV = 32
M = 2048
LOG_M = 11
LOG_TOTAL = 16
N_CROSS = 15
N_CHUNK = 2

CROSS_LD = [
    (L, dlog)
    for L in range(LOG_M + 1, LOG_TOTAL + 1)
    for dlog in range(L - 1, LOG_M - 1, -1)
]
assert len(CROSS_LD) == N_CROSS
LAST_OF_LEVEL = {k: L for k, (L, dlog) in enumerate(CROSS_LD) if dlog == LOG_M}


def kernel(x):
    m, n = x.shape
    assert m == M and n % N_CHUNK == 0
    cw = n // N_CHUNK

    def body(x_ref, out_ref, w0, w1, r0, r1, send_sems, recv_sems, ready_sems):
        my_pos = lax.axis_index("i")
        W = [w0, w1]
        R = [r0, r1]

        i2d = lax.broadcasted_iota(jnp.int32, (M, 1), 0)
        g = my_pos.astype(jnp.int32) * M + i2d

        def keep_min_mask(L, d):
            asc = ((g >> L) & 1) == 0
            bit = (g & d) != 0
            return bit, jnp.logical_xor(bit, asc)

        def local_stage(w, L, d):
            bit, keep_min = keep_min_mask(L, d)
            z = w[...]
            up = pltpu.roll(z, M - d, axis=0)
            dn = pltpu.roll(z, d, axis=0)
            partner = jnp.where(bit, dn, up)
            mn = jnp.minimum(z, partner)
            mx = jnp.maximum(z, partner)
            w[...] = jnp.where(keep_min, mn, mx)

        def local_sort(w):
            def level_body(L, _):
                def stage_body(j, _):
                    d = jnp.int32(1) << (L - 1 - j)
                    local_stage(w, L, d)
                    return 0
                return lax.fori_loop(0, jnp.minimum(L, LOG_M), stage_body, 0)

            lax.fori_loop(1, LOG_M + 1, level_body, 0)

        def local_merge(w, L):
            def stage_body(j, _):
                d = jnp.int32(1) << (LOG_M - 1 - j)
                local_stage(w, L, d)
                return 0

            lax.fori_loop(0, LOG_M, stage_body, 0)

        def grant(c, k):
            delta = 1 << (CROSS_LD[k][1] - LOG_M)
            pl.semaphore_signal(
                ready_sems.at[c, k],
                inc=1,
                device_id=(my_pos ^ delta,),
                device_id_type=pl.DeviceIdType.MESH,
            )

        def make_rdma(c, k):
            delta = 1 << (CROSS_LD[k][1] - LOG_M)
            return pltpu.make_async_remote_copy(
                src_ref=W[c],
                dst_ref=R[c],
                send_sem=send_sems.at[c],
                recv_sem=recv_sems.at[c],
                device_id=(my_pos ^ delta,),
                device_id_type=pl.DeviceIdType.MESH,
            )

        def ex_start(c, k):
            pl.semaphore_wait(ready_sems.at[c, k], 1)
            make_rdma(c, k).start()

        def ex_fin(c, k):
            make_rdma(c, k).wait()
            L, dlog = CROSS_LD[k]
            d = 1 << dlog
            _, keep_min = keep_min_mask(L, d)
            z = W[c][...]
            r = R[c][...]
            mn = jnp.minimum(z, r)
            mx = jnp.maximum(z, r)
            W[c][...] = jnp.where(keep_min, mn, mx)
            if k + 1 < N_CROSS:
                grant(c, k + 1)
            if k in LAST_OF_LEVEL:
                local_merge(W[c], LAST_OF_LEVEL[k])

        for c in range(N_CHUNK):
            grant(c, 0)

        w0[...] = x_ref[:, :cw].astype(jnp.bfloat16)
        local_sort(w0)
        ex_start(0, 0)
        w1[...] = x_ref[:, cw:].astype(jnp.bfloat16)
        local_sort(w1)
        ex_start(1, 0)
        for k in range(N_CROSS):
            for c in range(N_CHUNK):
                ex_fin(c, k)
                if k + 1 < N_CROSS:
                    ex_start(c, k + 1)

        out_ref[:, :cw] = w0[...].astype(jnp.float32)
        out_ref[:, cw:] = w1[...].astype(jnp.float32)

    return pl.pallas_call(
        body,
        out_shape=jax.ShapeDtypeStruct((m, n), jnp.float32),
        in_specs=[pl.BlockSpec(memory_space=pltpu.VMEM)],
        out_specs=pl.BlockSpec(memory_space=pltpu.VMEM),
        scratch_shapes=[
            pltpu.VMEM((M, cw), jnp.bfloat16),
            pltpu.VMEM((M, cw), jnp.bfloat16),
            pltpu.VMEM((M, cw), jnp.bfloat16),
            pltpu.VMEM((M, cw), jnp.bfloat16),
            pltpu.SemaphoreType.DMA((N_CHUNK,)),
            pltpu.SemaphoreType.DMA((N_CHUNK,)),
            pltpu.SemaphoreType.REGULAR((N_CHUNK, N_CROSS)),
        ],
    )(x)


# device time: 585245 ns/iter; 1.4470x vs baseline; 1.4470x over previous
import jax
import jax.numpy as jnp
from jax import lax
from jax.experimental import pallas as pl
from jax.experimental.pallas import tpu as pltpu

jax.config.update("jax_compilation_cache_dir", "/tmp/jax_cache")
jax.config.update("jax_persistent_cache_min_compile_time_secs", 0.0)

N_DEV = 32
M = 2048
LOG_M = 11
LOG_TOTAL = 16
N_CROSS = 15


def kernel(x):
    m, n = x.shape
    assert m == M

    cross_deltas = [
        1 << (dlog - LOG_M)
        for L in range(LOG_M + 1, LOG_TOTAL + 1)
        for dlog in range(L - 1, LOG_M - 1, -1)
    ]
    assert len(cross_deltas) == N_CROSS

    def body(x_ref, out_ref, wbuf, rbuf, send_sem, recv_sem, ready_sems):
        my_pos = lax.axis_index("i")

        wbuf[...] = x_ref[...].astype(jnp.bfloat16)

        i2d = lax.broadcasted_iota(jnp.int32, (M, 1), 0)
        g = my_pos.astype(jnp.int32) * M + i2d

        def grant_credit(k):
            pl.semaphore_signal(
                ready_sems.at[k],
                inc=1,
                device_id=(my_pos ^ cross_deltas[k],),
                device_id_type=pl.DeviceIdType.MESH,
            )

        grant_credit(0)

        cross_idx = 0
        for L in range(1, LOG_TOTAL + 1):
            for dlog in range(L - 1, -1, -1):
                d = 1 << dlog
                asc = ((g >> L) & 1) == 0
                bit = (g & d) != 0
                keep_min = jnp.logical_xor(bit, asc)
                if d < M:
                    z = wbuf[...]
                    up = pltpu.roll(z, M - d, axis=0)
                    dn = pltpu.roll(z, d, axis=0)
                    partner = jnp.where(bit, dn, up)
                    mn = jnp.minimum(z, partner)
                    mx = jnp.maximum(z, partner)
                    wbuf[...] = jnp.where(keep_min, mn, mx)
                else:
                    delta = d >> LOG_M
                    partner_dev = my_pos ^ delta
                    pl.semaphore_wait(ready_sems.at[cross_idx], 1)
                    rdma = pltpu.make_async_remote_copy(
                        src_ref=wbuf,
                        dst_ref=rbuf,
                        send_sem=send_sem,
                        recv_sem=recv_sem,
                        device_id=(partner_dev,),
                        device_id_type=pl.DeviceIdType.MESH,
                    )
                    rdma.start()
                    rdma.wait()
                    z = wbuf[...]
                    r = rbuf[...]
                    mn = jnp.minimum(z, r)
                    mx = jnp.maximum(z, r)
                    wbuf[...] = jnp.where(keep_min, mn, mx)
                    cross_idx += 1
                    if cross_idx < N_CROSS:
                        grant_credit(cross_idx)

        out_ref[...] = wbuf[...].astype(jnp.float32)

    return pl.pallas_call(
        body,
        out_shape=jax.ShapeDtypeStruct((m, n), jnp.float32),
        in_specs=[pl.BlockSpec(memory_space=pltpu.VMEM)],
        out_specs=pl.BlockSpec(memory_space=pltpu.VMEM),
        scratch_shapes=[
            pltpu.VMEM((M, n), jnp.bfloat16),
            pltpu.VMEM((M, n), jnp.bfloat16),
            pltpu.SemaphoreType.DMA,
            pltpu.SemaphoreType.DMA,
            pltpu.SemaphoreType.REGULAR((N_CROSS,)),
        ],
    )(x)
